# baseline (speedup 1.0000x reference)
import functools

import jax
import jax.numpy as jnp
from jax.experimental import pallas as pl

_EPS = 1e-5


def _ln(x, g, b):
    m1 = jnp.mean(x, axis=-1, keepdims=True)
    m2 = jnp.mean(x * x, axis=-1, keepdims=True)
    inv = jax.lax.rsqrt(m2 - m1 * m1 + _EPS)
    return (x - m1) * inv * g + b


def _dott(a, w):
    return jax.lax.dot_general(a, w, (((1,), (1,)), ((), ())),
                               preferred_element_type=jnp.float32)


def _fused_kernel(xf_ref, xe_ref, labc_ref, labr_ref, we_ref, wes_ref, be_ref,
                  wq_ref, bq_ref, wk_ref, bk_ref, wv_ref, bv_ref,
                  wo_ref, bo_ref, w1_ref, b1_ref, w2_ref, b2_ref,
                  ln1g_ref, ln1b_ref, ln2g_ref, ln2b_ref,
                  lng_ref, lnb_ref, wd_ref, bd_ref, out_ref,
                  *, n_var, seq, n_layer, n_tok, p_tok, pred):
    f32 = jnp.float32
    x = xf_ref[0]
    e = xe_ref[0]
    pad = jnp.zeros((4, seq), f32)
    hin = jnp.concatenate(
        [jnp.concatenate(
            [x[:, seq * v:seq * (v + 1)],
             e[:, seq * v:seq * (v + 1)], pad], axis=0)
         for v in range(n_var)], axis=0)
    hraw = _dott(hin, we_ref[...])
    mu = jnp.mean(x, axis=1, keepdims=True)
    var = jnp.mean((x - mu) ** 2, axis=1, keepdims=True)
    sig = jnp.sqrt(var + _EPS)
    inv = 1.0 / sig
    inv_p = jnp.concatenate([inv, jnp.ones((8, 1), f32)], axis=0)
    mui_p = jnp.concatenate([mu * inv, jnp.zeros((8, 1), f32)], axis=0)
    inv_full = jnp.concatenate([inv_p] * n_var, axis=0)
    mui_full = jnp.concatenate([mui_p] * n_var, axis=0)
    ht = inv_full * hraw - mui_full * wes_ref[...] + be_ref[...]

    mbias = jnp.where(labc_ref[0] == labr_ref[0], f32(0.0), f32(-1e9))
    for l in range(n_layer):
        q = _dott(ht, wq_ref[l]) + bq_ref[l]
        k = _dott(ht, wk_ref[l]) + bk_ref[l]
        v = _dott(ht, wv_ref[l]) + bv_ref[l]
        outs = []
        for vv in range(n_var):
            sl = slice(p_tok * vv, p_tok * (vv + 1))
            s = _dott(q[sl], k[sl])
            pex = jnp.exp(s + mbias)
            rsum = 1.0 / jnp.sum(pex, axis=-1, keepdims=True)
            outs.append(jnp.dot(pex, v[sl], preferred_element_type=f32) * rsum)
        o = jnp.concatenate(outs, axis=0)
        o = _dott(o, wo_ref[l]) + bo_ref[l]
        h2 = _ln(ht + o, ln1g_ref[l], ln1b_ref[l])
        f = _dott(h2, w1_ref[l]) + b1_ref[l]
        f = jnp.maximum(f, 0.0)
        f = _dott(f, w2_ref[l]) + b2_ref[l]
        ht = _ln(h2 + f, ln2g_ref[l], ln2b_ref[l])
    ho = _ln(ht, lng_ref[...], lnb_ref[...])
    dec = _dott(ho, wd_ref[...]) + bd_ref[...]
    for vv in range(n_var):
        out_ref[0, :, pred * vv:pred * (vv + 1)] = (
            dec[p_tok * vv:p_tok * vv + n_tok] * sig + mu)


def kernel(x, x_enc, We, be, Wq, bq, Wk, bk, Wv, bv, Wo, bo, W1, b1, W2, b2,
           ln1g, ln1b, ln2g, ln2b, lng, lnb, Wd, bd, R):
    B, N, V, S = x.shape
    d = We.shape[0]
    L = Wq.shape[0]
    dff = W1.shape[1]
    pred = Wd.shape[0]
    P = N + 8

    xf = x.reshape(B, N, V * S)
    proj = xf @ R
    bits = (proj > 0).astype(jnp.int32)
    labels = bits[..., 0] + 2 * bits[..., 1] + 4 * bits[..., 2]
    labf = labels.astype(jnp.float32)
    labp = jnp.concatenate(
        [labf, jnp.zeros((B, 4), jnp.float32), jnp.full((B, 4), -1.0, jnp.float32)],
        axis=1)
    labc = labp[:, :, None]
    labr = labp[:, None, :]
    xe = x_enc.reshape(B, 4, V * S)

    wfull = lambda shp: pl.BlockSpec(shp, lambda b: (0,) * len(shp))

    outp_call = pl.pallas_call(
        functools.partial(_fused_kernel, n_var=V, seq=S, n_layer=L, n_tok=N,
                          p_tok=P, pred=pred),
        grid=(B,),
        in_specs=[
            pl.BlockSpec((1, N, V * S), lambda b: (b, 0, 0)),
            pl.BlockSpec((1, 4, V * S), lambda b: (b, 0, 0)),
            pl.BlockSpec((1, P, 1), lambda b: (b, 0, 0)),
            pl.BlockSpec((1, 1, P), lambda b: (b, 0, 0)),
            wfull((d, S)),
            wfull((1, d)),
            wfull((1, d)),
            wfull((L, d, d)),
            wfull((L, 1, d)),
            wfull((L, d, d)),
            wfull((L, 1, d)),
            wfull((L, d, d)),
            wfull((L, 1, d)),
            wfull((L, d, d)),
            wfull((L, 1, d)),
            wfull((L, dff, d)),
            wfull((L, 1, dff)),
            wfull((L, d, dff)),
            wfull((L, 1, d)),
            wfull((L, 1, d)),
            wfull((L, 1, d)),
            wfull((L, 1, d)),
            wfull((L, 1, d)),
            wfull((1, d)),
            wfull((1, d)),
            wfull((pred, d)),
            wfull((1, pred)),
        ],
        out_specs=pl.BlockSpec((1, N, V * pred), lambda b: (b, 0, 0)),
        out_shape=jax.ShapeDtypeStruct((B, N, V * pred), jnp.float32),
    )
    scale = 1.0 / jnp.sqrt(jnp.float32(d))
    outp = outp_call(
      xf, xe, labc, labr, We, We.sum(axis=1)[None, :], be[None, :],
      Wq * scale, bq[:, None, :] * scale,
      Wk, bk[:, None, :],
      Wv, bv[:, None, :],
      Wo, bo[:, None, :],
      W1, b1[:, None, :],
      W2, b2[:, None, :],
      ln1g[:, None, :], ln1b[:, None, :], ln2g[:, None, :], ln2b[:, None, :],
      lng[None, :], lnb[None, :], Wd, bd[None, :])

    return outp.reshape(B, N, V, pred)


# speedup vs baseline: 1.0030x; 1.0030x over previous
import functools

import jax
import jax.numpy as jnp
from jax.experimental import pallas as pl

_EPS = 1e-5


def _ln(x, g, b):
    m1 = jnp.mean(x, axis=-1, keepdims=True)
    m2 = jnp.mean(x * x, axis=-1, keepdims=True)
    inv = jax.lax.rsqrt(m2 - m1 * m1 + _EPS)
    return (x - m1) * inv * g + b


def _dott(a, w):
    return jax.lax.dot_general(a, w, (((1,), (1,)), ((), ())),
                               preferred_element_type=jnp.float32)


def _fused_kernel(xf_ref, xe_ref, labc_ref, labr_ref, we_ref, be_ref,
                  wq_ref, bq_ref, wk_ref, bk_ref, wv_ref, bv_ref,
                  wo_ref, bo_ref, w1_ref, b1_ref, w2_ref, b2_ref,
                  ln1g_ref, ln1b_ref, ln2g_ref, ln2b_ref,
                  lng_ref, lnb_ref, wd_ref, bd_ref, out_ref,
                  *, n_var, seq, n_layer, n_tok, p_tok, pred):
    f32 = jnp.float32
    x = xf_ref[0]
    mu = jnp.mean(x, axis=1, keepdims=True)
    var = jnp.mean((x - mu) ** 2, axis=1, keepdims=True)
    sig = jnp.sqrt(var + _EPS)
    inv = 1.0 / sig
    e = xe_ref[0]
    pad = jnp.zeros((4, seq), f32)
    hin = jnp.concatenate(
        [jnp.concatenate(
            [(x[:, seq * v:seq * (v + 1)] - mu) * inv,
             e[:, seq * v:seq * (v + 1)], pad], axis=0)
         for v in range(n_var)], axis=0)
    ht = _dott(hin, we_ref[...]) + be_ref[...]

    mbias = jnp.where(labc_ref[0] == labr_ref[0], f32(0.0), f32(-1e9))
    for l in range(n_layer):
        q = _dott(ht, wq_ref[l]) + bq_ref[l]
        k = _dott(ht, wk_ref[l]) + bk_ref[l]
        v = _dott(ht, wv_ref[l]) + bv_ref[l]
        outs = []
        for vv in range(n_var):
            sl = slice(p_tok * vv, p_tok * (vv + 1))
            s = _dott(q[sl], k[sl])
            pex = jnp.exp(s + mbias)
            rsum = 1.0 / jnp.sum(pex, axis=-1, keepdims=True)
            outs.append(jnp.dot(pex, v[sl], preferred_element_type=f32) * rsum)
        o = jnp.concatenate(outs, axis=0)
        o = _dott(o, wo_ref[l]) + bo_ref[l]
        h2 = _ln(ht + o, ln1g_ref[l], ln1b_ref[l])
        f = _dott(h2, w1_ref[l]) + b1_ref[l]
        f = jnp.maximum(f, 0.0)
        f = _dott(f, w2_ref[l]) + b2_ref[l]
        ht = _ln(h2 + f, ln2g_ref[l], ln2b_ref[l])
    ho = _ln(ht, lng_ref[...], lnb_ref[...])
    dec = _dott(ho, wd_ref[...]) + bd_ref[...]
    for vv in range(n_var):
        out_ref[0, :, pred * vv:pred * (vv + 1)] = (
            dec[p_tok * vv:p_tok * vv + n_tok] * sig + mu)


def kernel(x, x_enc, We, be, Wq, bq, Wk, bk, Wv, bv, Wo, bo, W1, b1, W2, b2,
           ln1g, ln1b, ln2g, ln2b, lng, lnb, Wd, bd, R):
    B, N, V, S = x.shape
    d = We.shape[0]
    L = Wq.shape[0]
    dff = W1.shape[1]
    pred = Wd.shape[0]
    P = N + 8

    xf = x.reshape(B, N, V * S)
    proj = xf @ R
    bits = (proj > 0).astype(jnp.int32)
    labels = bits[..., 0] + 2 * bits[..., 1] + 4 * bits[..., 2]
    labf = labels.astype(jnp.float32)
    labp = jnp.concatenate(
        [labf, jnp.zeros((B, 4), jnp.float32), jnp.full((B, 4), -1.0, jnp.float32)],
        axis=1)
    labc = labp[:, :, None]
    labr = labp[:, None, :]
    xe = x_enc.reshape(B, 4, V * S)

    wfull = lambda shp: pl.BlockSpec(shp, lambda b: (0,) * len(shp))

    outp_call = pl.pallas_call(
        functools.partial(_fused_kernel, n_var=V, seq=S, n_layer=L, n_tok=N,
                          p_tok=P, pred=pred),
        grid=(B,),
        in_specs=[
            pl.BlockSpec((1, N, V * S), lambda b: (b, 0, 0)),
            pl.BlockSpec((1, 4, V * S), lambda b: (b, 0, 0)),
            pl.BlockSpec((1, P, 1), lambda b: (b, 0, 0)),
            pl.BlockSpec((1, 1, P), lambda b: (b, 0, 0)),
            wfull((d, S)),
            wfull((1, d)),
            wfull((L, d, d)),
            wfull((L, 1, d)),
            wfull((L, d, d)),
            wfull((L, 1, d)),
            wfull((L, d, d)),
            wfull((L, 1, d)),
            wfull((L, d, d)),
            wfull((L, 1, d)),
            wfull((L, dff, d)),
            wfull((L, 1, dff)),
            wfull((L, d, dff)),
            wfull((L, 1, d)),
            wfull((L, 1, d)),
            wfull((L, 1, d)),
            wfull((L, 1, d)),
            wfull((L, 1, d)),
            wfull((1, d)),
            wfull((1, d)),
            wfull((pred, d)),
            wfull((1, pred)),
        ],
        out_specs=pl.BlockSpec((1, N, V * pred), lambda b: (b, 0, 0)),
        out_shape=jax.ShapeDtypeStruct((B, N, V * pred), jnp.float32),
    )
    scale = 1.0 / jnp.sqrt(jnp.float32(d))
    outp = outp_call(
      xf, xe, labc, labr, We, be[None, :],
      Wq * scale, bq[:, None, :] * scale,
      Wk, bk[:, None, :],
      Wv, bv[:, None, :],
      Wo, bo[:, None, :],
      W1, b1[:, None, :],
      W2, b2[:, None, :],
      ln1g[:, None, :], ln1b[:, None, :], ln2g[:, None, :], ln2b[:, None, :],
      lng[None, :], lnb[None, :], Wd, bd[None, :])

    return outp.reshape(B, N, V, pred)
